# baseline (speedup 1.0000x reference)
import jax
import jax.numpy as jnp
from jax import lax
from jax.experimental import pallas as pl
from jax.experimental.pallas import tpu as pltpu
from jax.experimental.pallas import tpu_sc as plsc

_B = 4096
_T = 200
_D = 32
_V = 1000000
_N = _B * _T
_NC = 2
_NS = 16
_NW = _NC * _NS
_LANES = 16
_TO = 8
_NU = _T // _TO
_ROWS = _TO * 128
_R = _D // 8


def _body(xt_hbm, tab_hbm, pos_hbm, out_hbm, xv, rows_v, pos_v, sg, swb):
    wid = lax.axis_index("s") * _NC + lax.axis_index("c")
    c = wid

    pltpu.sync_copy(pos_hbm, pos_v)

    def gather_desc(ti):
        return pltpu.make_async_copy(
            tab_hbm.at[xv.at[ti]],
            rows_v.at[pl.ds(ti * 128, 128)],
            sg)

    def wb_drain():
        pltpu.make_async_copy(
            tab_hbm.at[pl.ds(0, _ROWS)], rows_v, swb).wait()

    def unit(tt, carry):
        @pl.when(tt > 0)
        def _():
            wb_drain()
        pltpu.sync_copy(
            xt_hbm.at[pl.ds(tt * _TO, _TO), pl.ds(c * 128, 128)], xv)
        for ti in range(_TO):
            pltpu.async_copy(
                tab_hbm.at[xv.at[ti]],
                rows_v.at[pl.ds(ti * 128, 128)],
                sg)

        def tbody(ti, carry2):
            gather_desc(ti).wait()
            t = tt * _TO + ti
            pv0 = pos_v[t, pl.ds(0, _LANES)]
            pv1 = pos_v[t, pl.ds(_LANES, _LANES)]
            rb = ti * 128

            def prow(k, carry3):
                for j in range(8):
                    row = rb + k * 8 + j
                    plsc.addupdate(rows_v.at[row, pl.ds(0, _LANES)], pv0)
                    plsc.addupdate(rows_v.at[row, pl.ds(_LANES, _LANES)], pv1)
                return carry3

            lax.fori_loop(0, 16, prow, 0)
            for r in range(_R):
                for s in range(8):
                    pltpu.async_copy(
                        rows_v.at[pl.ds(rb, 128), pl.ds(8 * r + s, 1)],
                        out_hbm.at[t, r, c, s],
                        swb)
            return carry2

        lax.fori_loop(0, _TO, tbody, 0)
        return carry

    lax.fori_loop(0, _NU, unit, 0)
    wb_drain()


def kernel(x, token_table, pos_table):
    xt = x.astype(jnp.int32).T
    mesh = plsc.VectorSubcoreMesh(core_axis_name="c", subcore_axis_name="s")
    l5 = pl.kernel(
        _body,
        out_type=jax.ShapeDtypeStruct((_T, _R, _NW, 8, 128, 1), jnp.float32),
        mesh=mesh,
        compiler_params=pltpu.CompilerParams(
            use_tc_tiling_on_sc=False, needs_layout_passes=False),
        scratch_types=[
            pltpu.VMEM((_TO, 128), jnp.int32),
            pltpu.VMEM((_ROWS, _D), jnp.float32),
            pltpu.VMEM((_T, _D), jnp.float32),
            pltpu.SemaphoreType.DMA,
            pltpu.SemaphoreType.DMA,
        ],
    )(xt, token_table, pos_table)
    return l5.reshape(_T, _R, _NW, 8, 128).transpose(
        (2, 4, 0, 1, 3)).reshape(_B, _T, _D)


# speedup vs baseline: 57.9674x; 57.9674x over previous
import jax
import jax.numpy as jnp
from jax import lax
from jax.experimental import pallas as pl
from jax.experimental.pallas import tpu as pltpu
from jax.experimental.pallas import tpu_sc as plsc

_B = 4096
_T = 200
_D = 32
_V = 1000000
_N = _B * _T
_NC = 2
_NS = 16
_NW = _NC * _NS
_LANES = 16
_TO = 8
_NU = _T // _TO
_ROWS = _TO * 128
_R = _D // 8


def _body(xt_hbm, tab_hbm, pos_hbm, out_hbm,
          xv, rows_v, slab_v, pos_v, sg, swb):
    wid = lax.axis_index("s") * _NC + lax.axis_index("c")
    c = wid

    pltpu.sync_copy(pos_hbm, pos_v)

    def gather_desc(ti):
        return pltpu.make_async_copy(
            tab_hbm.at[xv.at[ti]],
            rows_v.at[pl.ds(ti * 128, 128)],
            sg)

    def unit(tt, carry):
        pltpu.sync_copy(
            xt_hbm.at[pl.ds(tt * _TO, _TO), pl.ds(c * 128, 128)], xv)
        for ti in range(_TO):
            pltpu.async_copy(
                tab_hbm.at[xv.at[ti]],
                rows_v.at[pl.ds(ti * 128, 128)],
                sg)

        def tbody(ti, carry2):
            gather_desc(ti).wait()
            t = tt * _TO + ti
            par = lax.bitwise_and(ti, 1)
            @pl.when(t >= 2)
            def _():
                for r in range(_R):
                    pltpu.make_async_copy(
                        slab_v.at[par, r],
                        out_hbm.at[t, r, c],
                        swb).wait()
            bt = jnp.broadcast_to(t, (_LANES,))
            lrows = [
                ti * 128 + lg * _LANES + lax.iota(jnp.int32, _LANES)
                for lg in range(8)
            ]
            for r in range(_R):
                for s in range(8):
                    d = 8 * r + s
                    bc = jnp.full((_LANES,), d, jnp.int32)
                    pv = plsc.load_gather(pos_v, [bt, bc])
                    vs = [plsc.load_gather(rows_v, [lrows[lg], bc])
                          for lg in range(8)]
                    for lg in range(8):
                        slab_v[par, r, s, pl.ds(lg * _LANES, _LANES)] = (
                            vs[lg] + pv)
            for r in range(_R):
                pltpu.async_copy(slab_v.at[par, r], out_hbm.at[t, r, c], swb)
            return carry2

        lax.fori_loop(0, _TO, tbody, 0)
        return carry

    lax.fori_loop(0, _NU, unit, 0)
    for par, toff in ((0, 2), (1, 1)):
        t = _T - toff
        for r in range(_R):
            pltpu.make_async_copy(
                slab_v.at[par, r], out_hbm.at[t, r, c], swb).wait()


def kernel(x, token_table, pos_table):
    xt = x.astype(jnp.int32).T
    mesh = plsc.VectorSubcoreMesh(core_axis_name="c", subcore_axis_name="s")
    l5 = pl.kernel(
        _body,
        out_type=jax.ShapeDtypeStruct((_T, _R, _NW, 8, 128), jnp.float32),
        mesh=mesh,
        compiler_params=pltpu.CompilerParams(
            use_tc_tiling_on_sc=False, needs_layout_passes=False),
        scratch_types=[
            pltpu.VMEM((_TO, 128), jnp.int32),
            pltpu.VMEM((_ROWS, _D), jnp.float32),
            pltpu.VMEM((2, _R, 8, 128), jnp.float32),
            pltpu.VMEM((_T, _D), jnp.float32),
            pltpu.SemaphoreType.DMA,
            pltpu.SemaphoreType.DMA,
        ],
    )(xt, token_table, pos_table)
    return l5.transpose((2, 4, 0, 1, 3)).reshape(_B, _T, _D)


# speedup vs baseline: 59.1425x; 1.0203x over previous
import jax
import jax.numpy as jnp
from jax import lax
from jax.experimental import pallas as pl
from jax.experimental.pallas import tpu as pltpu
from jax.experimental.pallas import tpu_sc as plsc

_B = 4096
_T = 200
_D = 32
_V = 1000000
_N = _B * _T
_NC = 2
_NS = 16
_NW = _NC * _NS
_LANES = 16
_TO = 8
_NU = _T // _TO
_ROWS = _TO * 128
_R = _D // 8


def _body(xt_hbm, tab_hbm, pos_hbm, out_hbm,
          xv, rows_v, slab_v, pos_v, sg, swb):
    wid = lax.axis_index("s") * _NC + lax.axis_index("c")
    c = wid

    pltpu.sync_copy(pos_hbm, pos_v)

    def fire_unit(tt, pu):
        pltpu.sync_copy(
            xt_hbm.at[pl.ds(tt * _TO, _TO), pl.ds(c * 128, 128)], xv.at[pu])
        for ti in range(_TO):
            pltpu.async_copy(
                tab_hbm.at[xv.at[pu, ti]],
                rows_v.at[pu, pl.ds(ti * 128, 128)],
                sg.at[pu])

    fire_unit(0, 0)

    def unit(tt, carry):
        pu = lax.bitwise_and(tt, 1)
        pn = 1 - pu

        @pl.when(tt + 1 < _NU)
        def _():
            fire_unit(tt + 1, pn)

        def tbody(ti, carry2):
            pltpu.make_async_copy(
                tab_hbm.at[xv.at[pu, ti]],
                rows_v.at[pu, pl.ds(ti * 128, 128)],
                sg.at[pu]).wait()
            t = tt * _TO + ti
            par = lax.bitwise_and(ti, 1)
            @pl.when(t >= 2)
            def _():
                for r in range(_R):
                    pltpu.make_async_copy(
                        slab_v.at[par, r],
                        out_hbm.at[t, r, c],
                        swb).wait()
            bt = jnp.broadcast_to(t, (_LANES,))
            lrows = [
                ti * 128 + lg * _LANES + lax.iota(jnp.int32, _LANES)
                for lg in range(8)
            ]
            for r in range(_R):
                for s in range(8):
                    d = 8 * r + s
                    bc = jnp.full((_LANES,), d, jnp.int32)
                    pv = plsc.load_gather(pos_v, [bt, bc])
                    vs = [plsc.load_gather(rows_v.at[pu], [lrows[lg], bc])
                          for lg in range(8)]
                    for lg in range(8):
                        slab_v[par, r, s, pl.ds(lg * _LANES, _LANES)] = (
                            vs[lg] + pv)
            for r in range(_R):
                pltpu.async_copy(slab_v.at[par, r], out_hbm.at[t, r, c], swb)
            return carry2

        lax.fori_loop(0, _TO, tbody, 0)
        return carry

    lax.fori_loop(0, _NU, unit, 0)
    for par, toff in ((0, 2), (1, 1)):
        t = _T - toff
        for r in range(_R):
            pltpu.make_async_copy(
                slab_v.at[par, r], out_hbm.at[t, r, c], swb).wait()


def kernel(x, token_table, pos_table):
    xt = x.astype(jnp.int32).T
    mesh = plsc.VectorSubcoreMesh(core_axis_name="c", subcore_axis_name="s")
    l5 = pl.kernel(
        _body,
        out_type=jax.ShapeDtypeStruct((_T, _R, _NW, 8, 128), jnp.float32),
        mesh=mesh,
        compiler_params=pltpu.CompilerParams(
            use_tc_tiling_on_sc=False, needs_layout_passes=False),
        scratch_types=[
            pltpu.VMEM((2, _TO, 128), jnp.int32),
            pltpu.VMEM((2, _ROWS, _D), jnp.float32),
            pltpu.VMEM((2, _R, 8, 128), jnp.float32),
            pltpu.VMEM((_T, _D), jnp.float32),
            pltpu.SemaphoreType.DMA((2,)),
            pltpu.SemaphoreType.DMA,
        ],
    )(xt, token_table, pos_table)
    return l5.transpose((2, 4, 0, 1, 3)).reshape(_B, _T, _D)
